# Initial kernel scaffold; baseline (speedup 1.0000x reference)
import jax
import jax.numpy as jnp
from jax.experimental import pallas as pl


def kernel(inputs, table):
    raise NotImplementedError("write your pallas kernel here")


# speedup vs baseline: 1.0938x; 1.0938x over previous
import functools

import jax
import jax.numpy as jnp
from jax import lax
from jax.experimental import pallas as pl
from jax.experimental.pallas import tpu as pltpu
from jax.experimental.pallas import tpu_sc as plsc

_NUM_CORES = 2
_NUM_SUBCORES = 16
_NUM_WORKERS = _NUM_CORES * _NUM_SUBCORES


@functools.partial(jax.jit, static_argnums=(2, 3, 4))
def _sc_gather(flat_idx, table, N, D, CH):
    per_w = N // _NUM_WORKERS
    n_chunks = per_w // CH
    mesh = plsc.VectorSubcoreMesh(
        core_axis_name="c", subcore_axis_name="s",
        num_cores=_NUM_CORES, num_subcores=_NUM_SUBCORES)

    @functools.partial(
        pl.kernel,
        out_type=jax.ShapeDtypeStruct((N, D), jnp.float32),
        mesh=mesh,
        scratch_types=[
            pltpu.VMEM((CH,), jnp.int32),
            pltpu.VMEM((CH, D), jnp.float32),
            pltpu.SemaphoreType.DMA,
        ],
        compiler_params=pltpu.CompilerParams(use_tc_tiling_on_sc=False),
    )
    def k(idx_hbm, table_hbm, out_hbm, idx_v, rows_v, sem):
        wid = lax.axis_index("s") * _NUM_CORES + lax.axis_index("c")
        base = wid * per_w

        def body(i, carry):
            off = base + i * CH
            pltpu.sync_copy(idx_hbm.at[pl.ds(off, CH)], idx_v)
            pltpu.async_copy(table_hbm.at[idx_v], rows_v, sem).wait()
            pltpu.sync_copy(rows_v, out_hbm.at[pl.ds(off, CH)])
            return carry

        lax.fori_loop(0, n_chunks, body, 0, unroll=False)

    return k(flat_idx, table)


def kernel(inputs, table):
    B, H = inputs.shape
    V, D = table.shape
    N = B * H
    flat_idx = inputs.reshape(N).astype(jnp.int32)
    CH = 1024
    out = _sc_gather(flat_idx, table, N, D, CH)
    return out.reshape(B, H, D)


# speedup vs baseline: 1.1118x; 1.0165x over previous
import functools

import jax
import jax.numpy as jnp
from jax import lax
from jax.experimental import pallas as pl
from jax.experimental.pallas import tpu as pltpu
from jax.experimental.pallas import tpu_sc as plsc

_NUM_CORES = 2
_NUM_SUBCORES = 16
_NUM_WORKERS = _NUM_CORES * _NUM_SUBCORES


@functools.partial(jax.jit, static_argnums=(2, 3, 4))
def _sc_gather(flat_idx, table, N, D, CH):
    per_w = N // _NUM_WORKERS
    n_chunks = per_w // CH
    mesh = plsc.VectorSubcoreMesh(
        core_axis_name="c", subcore_axis_name="s",
        num_cores=_NUM_CORES, num_subcores=_NUM_SUBCORES)

    @functools.partial(
        pl.kernel,
        out_type=jax.ShapeDtypeStruct((N, D), jnp.float32),
        mesh=mesh,
        scratch_types=[
            pltpu.VMEM((CH,), jnp.int32),
            pltpu.VMEM((CH,), jnp.int32),
            pltpu.VMEM((CH, D), jnp.float32),
            pltpu.VMEM((CH, D), jnp.float32),
            pltpu.SemaphoreType.DMA,
            pltpu.SemaphoreType.DMA,
            pltpu.SemaphoreType.DMA,
            pltpu.SemaphoreType.DMA,
            pltpu.SemaphoreType.DMA,
            pltpu.SemaphoreType.DMA,
        ],
        compiler_params=pltpu.CompilerParams(use_tc_tiling_on_sc=False),
    )
    def k(idx_hbm, table_hbm, out_hbm,
          idx0, idx1, rows0, rows1, si0, si1, sg0, sg1, ss0, ss1):
        wid = lax.axis_index("s") * _NUM_CORES + lax.axis_index("c")
        base = wid * per_w
        idx_v = (idx0, idx1)
        rows_v = (rows0, rows1)
        si = (si0, si1)
        sg = (sg0, sg1)
        ss = (ss0, ss1)

        def start_idx(i, s):
            pltpu.async_copy(idx_hbm.at[pl.ds(base + i * CH, CH)],
                             idx_v[s], si[s])

        def wait_idx(s):
            pltpu.make_async_copy(idx_hbm.at[pl.ds(base, CH)],
                                  idx_v[s], si[s]).wait()

        def start_gather(s):
            pltpu.async_copy(table_hbm.at[idx_v[s]], rows_v[s], sg[s])

        def wait_gather(s):
            pltpu.make_async_copy(table_hbm.at[idx_v[s]],
                                  rows_v[s], sg[s]).wait()

        def start_store(i, s):
            pltpu.async_copy(rows_v[s],
                             out_hbm.at[pl.ds(base + i * CH, CH)], ss[s])

        def wait_store(s):
            pltpu.make_async_copy(rows_v[s],
                                  out_hbm.at[pl.ds(base, CH)], ss[s]).wait()

        start_idx(0, 0)
        if n_chunks > 1:
            start_idx(1, 1)
        wait_idx(0)
        start_gather(0)
        if n_chunks > 1:
            wait_idx(1)
            start_gather(1)

        for i in range(n_chunks):
            s = i % 2
            wait_gather(s)
            start_store(i, s)
            if i + 2 < n_chunks:
                start_idx(i + 2, s)
                wait_idx(s)
                wait_store(s)
                start_gather(s)

        wait_store(n_chunks % 2)
        if n_chunks > 1:
            wait_store((n_chunks - 1) % 2)

    return k(flat_idx, table)


def kernel(inputs, table):
    B, H = inputs.shape
    V, D = table.shape
    N = B * H
    flat_idx = inputs.reshape(N).astype(jnp.int32)
    CH = 1600
    out = _sc_gather(flat_idx, table, N, D, CH)
    return out.reshape(B, H, D)


# speedup vs baseline: 1.8004x; 1.6193x over previous
import functools

import jax
import jax.numpy as jnp
from jax import lax
from jax.experimental import pallas as pl
from jax.experimental.pallas import tpu as pltpu
from jax.experimental.pallas import tpu_sc as plsc

_NUM_CORES = 2
_NUM_SUBCORES = 16
_NUM_WORKERS = _NUM_CORES * _NUM_SUBCORES


@functools.partial(jax.jit, static_argnums=(2, 3, 4, 5))
def _sc_embed(idx2d, table, B, H, D, B_CH):
    per_w = B // _NUM_WORKERS
    n_blocks = per_w // B_CH
    mesh = plsc.VectorSubcoreMesh(
        core_axis_name="c", subcore_axis_name="s",
        num_cores=_NUM_CORES, num_subcores=_NUM_SUBCORES)

    @functools.partial(
        pl.kernel,
        out_type=jax.ShapeDtypeStruct((B, H, D), jnp.float32),
        mesh=mesh,
        scratch_types=[
            pltpu.VMEM((B_CH, H), jnp.int32),
            pltpu.VMEM((B_CH, H), jnp.int32),
            pltpu.VMEM((B_CH, H, D), jnp.float32),
            pltpu.VMEM((B_CH, H, D), jnp.float32),
            pltpu.SemaphoreType.DMA,
            pltpu.SemaphoreType.DMA,
            pltpu.SemaphoreType.DMA,
            pltpu.SemaphoreType.DMA,
            pltpu.SemaphoreType.DMA,
            pltpu.SemaphoreType.DMA,
        ],
        compiler_params=pltpu.CompilerParams(use_tc_tiling_on_sc=False),
    )
    def k(idx_hbm, table_hbm, out_hbm,
          idx0, idx1, rows0, rows1, si0, si1, sg0, sg1, ss0, ss1):
        wid = lax.axis_index("s") * _NUM_CORES + lax.axis_index("c")
        base = wid * per_w
        idx_v = (idx0, idx1)
        rows_v = (rows0, rows1)
        si = (si0, si1)
        sg = (sg0, sg1)
        ss = (ss0, ss1)

        def start_idx(i, s):
            pltpu.async_copy(idx_hbm.at[pl.ds(base + i * B_CH, B_CH)],
                             idx_v[s], si[s])

        def wait_idx(s):
            pltpu.make_async_copy(idx_hbm.at[pl.ds(base, B_CH)],
                                  idx_v[s], si[s]).wait()

        def start_gathers(s):
            for b in range(B_CH):
                pltpu.async_copy(table_hbm.at[idx_v[s].at[b]],
                                 rows_v[s].at[b], sg[s])

        def wait_gathers(s):
            for b in range(B_CH):
                pltpu.make_async_copy(table_hbm.at[idx_v[s].at[b]],
                                      rows_v[s].at[b], sg[s]).wait()

        def start_store(i, s):
            pltpu.async_copy(rows_v[s],
                             out_hbm.at[pl.ds(base + i * B_CH, B_CH)], ss[s])

        def wait_store(s):
            pltpu.make_async_copy(rows_v[s],
                                  out_hbm.at[pl.ds(base, B_CH)], ss[s]).wait()

        start_idx(0, 0)
        start_idx(1, 1)
        wait_idx(0)
        start_gathers(0)
        wait_idx(1)
        start_gathers(1)

        def body(p, carry):
            for s in range(2):
                i = 2 * p + s
                wait_gathers(s)
                start_store(i, s)
                start_idx(i + 2, s)
                wait_idx(s)
                wait_store(s)
                start_gathers(s)
            return carry

        lax.fori_loop(0, n_blocks // 2 - 1, body, 0, unroll=False)

        for s in range(2):
            i = n_blocks - 2 + s
            wait_gathers(s)
            start_store(i, s)
        wait_store(0)
        wait_store(1)

    return k(idx2d, table)


def kernel(inputs, table):
    B, H = inputs.shape
    V, D = table.shape
    idx2d = inputs.astype(jnp.int32)
    B_CH = 16
    return _sc_embed(idx2d, table, B, H, D, B_CH)
